# baseline (speedup 1.0000x reference)
import functools

import jax
import jax.numpy as jnp
from jax import lax
from jax.experimental import pallas as pl
from jax.experimental.pallas import tpu as pltpu
from jax.experimental.pallas import tpu_sc as plsc

N = 10000
E = 320000
H = 128
G = 16
NC = 2
NS = 16
NW = NC * NS
EPT = E // NW
K = 80
NWIN = EPT // K
RPS = N // NS
NPAD = 10240

_mesh = plsc.VectorSubcoreMesh(core_axis_name="c", subcore_axis_name="s")


@functools.partial(
    pl.kernel,
    mesh=_mesh,
    out_type=jax.ShapeDtypeStruct((NC, NPAD), jnp.float32),
    scratch_types=[
        pltpu.VMEM((NWIN, K), jnp.int32),
        pltpu.VMEM((NWIN, K), jnp.float32),
        pltpu.VMEM((NPAD // NS,), jnp.float32),
        pltpu.VMEM_SHARED((NPAD,), jnp.float32),
    ],
)
def _deg_kernel(col_hbm, w_hbm, out_hbm, colv, wv, zerov, accsh):
    c = lax.axis_index("c")
    s = lax.axis_index("s")
    wid = s * NC + c
    pltpu.sync_copy(col_hbm.at[wid], colv)
    pltpu.sync_copy(w_hbm.at[wid], wv)
    zslice = NPAD // NS
    for u in range(zslice // 16):
        zerov[pl.ds(u * 16, 16)] = jnp.zeros((16,), jnp.float32)
    pltpu.sync_copy(zerov, accsh.at[pl.ds(s * zslice, zslice)])
    plsc.subcore_barrier()

    def body(j, carry):
        pltpu.sync_copy(wv.at[j], accsh.at[colv.at[j]], add=True)
        return carry

    lax.fori_loop(0, NWIN, body, 0)
    plsc.subcore_barrier()
    pltpu.sync_copy(accsh.at[pl.ds(s * zslice, zslice)],
                    out_hbm.at[c, pl.ds(s * zslice, zslice)])


@functools.partial(
    pl.kernel,
    mesh=_mesh,
    out_type=jax.ShapeDtypeStruct((NC, NPAD, H), jnp.float32),
    scratch_types=[
        pltpu.VMEM((NWIN // 5, K), jnp.int32),
        pltpu.VMEM((NWIN // 5, K), jnp.int32),
        pltpu.VMEM((NWIN // 5, K), jnp.float32),
        pltpu.VMEM((3, K, H), jnp.float32),
        pltpu.VMEM((8, H), jnp.float32),
        pltpu.VMEM_SHARED((NPAD, H), jnp.float32),
        pltpu.SemaphoreType.DMA,
        pltpu.SemaphoreType.DMA,
    ],
)
def _scatter_kernel(hws_hbm, row_hbm, col_hbm, w_hbm, out_hbm,
                    rowv, colv, wv, rows3, zeros, accsh, gsem, ssem):
    c = lax.axis_index("c")
    s = lax.axis_index("s")
    wid = s * NC + c
    CW = NWIN // 5
    for j in range(8):
        for u in range(H // 16):
            zeros[j, pl.ds(u * 16, 16)] = jnp.zeros((16,), jnp.float32)
    rps = NPAD // NS
    for i in range(rps // 8):
        pltpu.sync_copy(zeros, accsh.at[pl.ds(s * rps + i * 8, 8)])
    plsc.subcore_barrier()

    def wait_gather(j, bi):
        pltpu.make_async_copy(hws_hbm.at[rowv.at[j]], rows3.at[bi], gsem).wait()

    def start_gather(j, bi):
        pltpu.async_copy(hws_hbm.at[rowv.at[j]], rows3.at[bi], gsem)

    def wait_scatter(j, bi):
        pltpu.make_async_copy(rows3.at[bi], accsh.at[colv.at[j]], ssem).wait()

    def start_scatter(j, bi):
        pltpu.async_copy(rows3.at[bi], accsh.at[colv.at[j]], ssem, add=True)

    def scale(j, bi):
        for b in range(K // 16):
            wchunk = wv[j, pl.ds(b * 16, 16)]
            for l in range(16):
                i = b * 16 + l
                wj = wchunk[l]
                for u in range(H // 16):
                    rows3[bi, i, pl.ds(u * 16, 16)] = (
                        rows3[bi, i, pl.ds(u * 16, 16)] * wj)

    def step(j, bi, g, guarded):
        wait_gather(j, bi)
        if guarded:
            @pl.when(g > 0)
            def _():
                wait_scatter(j - 1, (bi + 2) % 3)
        else:
            wait_scatter(j - 1, (bi + 2) % 3)

        @pl.when(j + 2 < CW)
        def _():
            start_gather(j + 2, (bi + 2) % 3)
        scale(j, bi)
        start_scatter(j, bi)

    def group(g, carry):
        j0 = 3 * g
        step(j0, 0, g, True)
        step(j0 + 1, 1, g, False)
        step(j0 + 2, 2, g, False)
        return carry

    def chunk_body(chunk, carry):
        pltpu.sync_copy(row_hbm.at[wid, chunk], rowv)
        pltpu.sync_copy(col_hbm.at[wid, chunk], colv)
        pltpu.sync_copy(w_hbm.at[wid, chunk], wv)
        start_gather(0, 0)
        start_gather(1, 1)
        lax.fori_loop(0, (CW - 1) // 3, group, 0)
        wait_gather(CW - 1, 0)
        wait_scatter(CW - 2, 2)
        scale(CW - 1, 0)
        start_scatter(CW - 1, 0)
        wait_scatter(CW - 1, 0)
        return carry

    lax.fori_loop(0, 5, chunk_body, 0)
    plsc.subcore_barrier()
    for i in range(rps // 128):
        pltpu.sync_copy(accsh.at[pl.ds(s * rps + i * 128, 128)],
                        out_hbm.at[c, pl.ds(s * rps + i * 128, 128)])


_B = 1000


def _embed_body(xr, d0r, d1r, gr, br, Wer, ber, W1r, hws_out, dis_out):
    deg = d0r[...] + d1r[...] + 1.0
    dis = lax.rsqrt(deg)
    xb = xr[...]
    mu = jnp.mean(xb, axis=1, keepdims=True)
    va = jnp.mean((xb - mu) * (xb - mu), axis=1, keepdims=True)
    ln = (xb - mu) * lax.rsqrt(va + 1e-5) * gr[...] + br[...]
    h0 = jnp.dot(ln, Wer[...], preferred_element_type=jnp.float32) + ber[...]
    hws_out[...] = jnp.dot(h0, W1r[...], preferred_element_type=jnp.float32) * dis
    dis_out[...] = dis


def _mid_body(a0r, a1r, hr, disr, br, Wr, out):
    dis = disr[...]
    h = jnp.maximum(dis * (a0r[...] + a1r[...] + hr[...]) + br[...], 0.0)
    out[...] = jnp.dot(h, Wr[...], preferred_element_type=jnp.float32) * dis


def _final_body(a0r, a1r, hr, disr, br, xr, batchr, gfr, Wgr, bgr,
                W1av, W1cv, W1gv, bh1r, Wh2r, bh2r, qout,
                sv, sc_, cv, cc_):
    i = pl.program_id(0)

    @pl.when(i == 0)
    def _():
        sv[...] = jnp.zeros_like(sv)
        sc_[...] = jnp.zeros_like(sc_)
        cv[...] = jnp.zeros_like(cv)
        cc_[...] = jnp.zeros_like(cc_)

    dis = disr[...]
    h = jnp.maximum(dis * (a0r[...] + a1r[...] + hr[...]) + br[...], 0.0)
    xb = xr[...]
    mv = (xb[:, H - 2:H - 1] > 0.5).astype(jnp.float32)
    mc = (xb[:, H - 1:H] > 0.5).astype(jnp.float32)
    onehot = (batchr[...] == lax.broadcasted_iota(jnp.int32, (_B, G), 1)
              ).astype(jnp.float32)
    Sv = onehot * mv
    Sc = onehot * mc
    dn = (((0,), (0,)), ((), ()))
    sv[...] += lax.dot_general(Sv, h, dn, preferred_element_type=jnp.float32)
    sc_[...] += lax.dot_general(Sc, h, dn, preferred_element_type=jnp.float32)
    ones = jnp.ones((_B, 1), jnp.float32)
    cv[...] += lax.dot_general(Sv, ones, dn, preferred_element_type=jnp.float32)
    cc_[...] += lax.dot_general(Sc, ones, dn, preferred_element_type=jnp.float32)

    @pl.when(i == pl.num_programs(0) - 1)
    def _():
        ve = sv[...] / jnp.clip(cv[...], 1.0, None)
        ce = sc_[...] / jnp.clip(cc_[...], 1.0, None)
        ge = jnp.maximum(
            jnp.dot(gfr[...], Wgr[...], preferred_element_type=jnp.float32)
            + bgr[...], 0.0)
        comb = jnp.maximum(
            jnp.dot(ve, W1av[...], preferred_element_type=jnp.float32)
            + jnp.dot(ce, W1cv[...], preferred_element_type=jnp.float32)
            + jnp.dot(ge, W1gv[...], preferred_element_type=jnp.float32)
            + bh1r[...], 0.0)
        qout[...] = (jnp.dot(comb, Wh2r[...], preferred_element_type=jnp.float32)
                     + bh2r[...])


def _row_spec(shape):
    return pl.BlockSpec(shape, lambda i: (i, 0))


def _full_spec(shape):
    return pl.BlockSpec(shape, lambda i: (0, 0))


def kernel(x, edge_index, batch, global_features, edge_weight,
           ln_g, ln_b, W_emb, b_emb, W1, b1, W2, b2, W3, b3,
           Wg, bg, Wh1, bh1, Wh2, bh2):
    f32 = jnp.float32
    row2d = edge_index[0].reshape(NW, 5, NWIN // 5, K)
    col2d = edge_index[1].reshape(NW, 5, NWIN // 5, K)
    w2d = edge_weight.reshape(NW, 5, NWIN // 5, K)

    degp = _deg_kernel(edge_index[1].reshape(NW, NWIN, K),
                       edge_weight.reshape(NW, NWIN, K))
    d0 = degp[0, :N].reshape(N, 1)
    d1 = degp[1, :N].reshape(N, 1)

    grid = N // _B
    hws1, dis = pl.pallas_call(
        _embed_body,
        grid=(grid,),
        in_specs=[_row_spec((_B, H)), _row_spec((_B, 1)), _row_spec((_B, 1)),
                  _full_spec((1, H)), _full_spec((1, H)),
                  _full_spec((H, H)), _full_spec((1, H)), _full_spec((H, H))],
        out_specs=[_row_spec((_B, H)), _row_spec((_B, 1))],
        out_shape=[jax.ShapeDtypeStruct((N, H), f32),
                   jax.ShapeDtypeStruct((N, 1), f32)],
    )(x, d0, d1, ln_g.reshape(1, H), ln_b.reshape(1, H),
      W_emb, b_emb.reshape(1, H), W1)

    def mid(hws, b_prev, W_next):
        accp = _scatter_kernel(hws, row2d, col2d, w2d)
        acc = accp[:, :N]
        return pl.pallas_call(
            _mid_body,
            grid=(grid,),
            in_specs=[_row_spec((_B, H)), _row_spec((_B, H)),
                      _row_spec((_B, H)), _row_spec((_B, 1)),
                      _full_spec((1, H)), _full_spec((H, H))],
            out_specs=_row_spec((_B, H)),
            out_shape=jax.ShapeDtypeStruct((N, H), f32),
        )(acc[0], acc[1], hws, dis, b_prev.reshape(1, H), W_next)

    hws2 = mid(hws1, b1, W2)
    hws3 = mid(hws2, b2, W3)
    acc3 = _scatter_kernel(hws3, row2d, col2d, w2d)[:, :N]

    q = pl.pallas_call(
        _final_body,
        grid=(grid,),
        in_specs=[_row_spec((_B, H)), _row_spec((_B, H)), _row_spec((_B, H)),
                  _row_spec((_B, 1)), _full_spec((1, H)), _row_spec((_B, H)),
                  _row_spec((_B, 1)),
                  _full_spec((G, 64)), _full_spec((64, H)), _full_spec((1, H)),
                  _full_spec((H, H)), _full_spec((H, H)), _full_spec((H, H)),
                  _full_spec((1, H)), _full_spec((H, 32)), _full_spec((1, 32))],
        out_specs=_full_spec((G, 32)),
        out_shape=jax.ShapeDtypeStruct((G, 32), f32),
        scratch_shapes=[pltpu.VMEM((G, H), f32), pltpu.VMEM((G, H), f32),
                        pltpu.VMEM((G, 1), f32), pltpu.VMEM((G, 1), f32)],
    )(acc3[0], acc3[1], hws3, dis, b3.reshape(1, H), x,
      batch.reshape(N, 1), global_features, Wg, bg.reshape(1, H),
      Wh1[:H], Wh1[H:2 * H], Wh1[2 * H:], bh1.reshape(1, H), Wh2,
      bh2.reshape(1, 32))
    return q


# speedup vs baseline: 1.1660x; 1.1660x over previous
import functools

import jax
import jax.numpy as jnp
from jax import lax
from jax.experimental import pallas as pl
from jax.experimental.pallas import tpu as pltpu
from jax.experimental.pallas import tpu_sc as plsc

N = 10000
E = 320000
H = 128
G = 16
NC = 2
NS = 16
NW = NC * NS
EPT = E // NW
K = 80
NWIN = EPT // K
RPS = N // NS
NPAD = 10240

_mesh = plsc.VectorSubcoreMesh(core_axis_name="c", subcore_axis_name="s")


@functools.partial(
    pl.kernel,
    mesh=_mesh,
    out_type=jax.ShapeDtypeStruct((NC, NPAD), jnp.float32),
    scratch_types=[
        pltpu.VMEM((NWIN, K), jnp.int32),
        pltpu.VMEM((NWIN, K), jnp.float32),
        pltpu.VMEM((NPAD // NS,), jnp.float32),
        pltpu.VMEM_SHARED((NPAD,), jnp.float32),
    ],
)
def _deg_kernel(col_hbm, w_hbm, out_hbm, colv, wv, zerov, accsh):
    c = lax.axis_index("c")
    s = lax.axis_index("s")
    wid = s * NC + c
    pltpu.sync_copy(col_hbm.at[wid], colv)
    pltpu.sync_copy(w_hbm.at[wid], wv)
    zslice = NPAD // NS
    for u in range(zslice // 16):
        zerov[pl.ds(u * 16, 16)] = jnp.zeros((16,), jnp.float32)
    pltpu.sync_copy(zerov, accsh.at[pl.ds(s * zslice, zslice)])
    plsc.subcore_barrier()

    def body(j, carry):
        pltpu.sync_copy(wv.at[j], accsh.at[colv.at[j]], add=True)
        return carry

    lax.fori_loop(0, NWIN, body, 0)
    plsc.subcore_barrier()
    pltpu.sync_copy(accsh.at[pl.ds(s * zslice, zslice)],
                    out_hbm.at[c, pl.ds(s * zslice, zslice)])


@functools.partial(
    pl.kernel,
    mesh=_mesh,
    out_type=jax.ShapeDtypeStruct((NC, NPAD, H), jnp.float32),
    compiler_params=pltpu.CompilerParams(needs_layout_passes=False,
                                         use_tc_tiling_on_sc=False),
    scratch_types=[
        pltpu.VMEM((NWIN // 5, K), jnp.int32),
        pltpu.VMEM((NWIN // 5, K), jnp.int32),
        pltpu.VMEM((NWIN // 5, K), jnp.float32),
        pltpu.VMEM((2, K, H // 2), jnp.int32),
        pltpu.VMEM((2, K, H), jnp.float32),
        pltpu.VMEM((16, H), jnp.float32),
        pltpu.VMEM_SHARED((NPAD, H), jnp.float32),
        pltpu.SemaphoreType.DMA,
        pltpu.SemaphoreType.DMA,
    ],
)
def _scatter_kernel(hwsb_hbm, row_hbm, col_hbm, w_hbm, out_hbm,
                    rowv, colv, wv, rowsb, rowsf, zeros, accsh, gsem, ssem):
    c = lax.axis_index("c")
    s = lax.axis_index("s")
    wid = s * NC + c
    CW = NWIN // 5
    for j in range(16):
        for u in range(H // 16):
            zeros[j, pl.ds(u * 16, 16)] = jnp.zeros((16,), jnp.float32)
    rps = NPAD // NS
    for i in range(rps // 16):
        pltpu.sync_copy(zeros, accsh.at[pl.ds(s * rps + i * 16, 16)])
    plsc.subcore_barrier()

    def wait_gather(j, bi):
        pltpu.make_async_copy(hwsb_hbm.at[rowv.at[j]], rowsb.at[bi], gsem).wait()

    def start_gather(j, bi):
        pltpu.async_copy(hwsb_hbm.at[rowv.at[j]], rowsb.at[bi], gsem)

    def wait_scatter(j, bi):
        pltpu.make_async_copy(rowsf.at[bi], accsh.at[colv.at[j]], ssem).wait()

    def start_scatter(j, bi):
        pltpu.async_copy(rowsf.at[bi], accsh.at[colv.at[j]], ssem, add=True)

    def scale(j, bi):
        himask = jnp.full((16,), -65536, jnp.int32)
        for b in range(K // 16):
            wchunk = wv[j, pl.ds(b * 16, 16)]
            for l in range(16):
                i = b * 16 + l
                wj = wchunk[l]
                for u in range(H // 32):
                    v = rowsb[bi, i, pl.ds(u * 16, 16)]
                    lo = plsc.bitcast(v << 16, jnp.float32)
                    hi = plsc.bitcast(v & himask, jnp.float32)
                    rowsf[bi, i, pl.ds(u * 32, 16)] = lo * wj
                    rowsf[bi, i, pl.ds(u * 32 + 16, 16)] = hi * wj

    def step(j, bi, g, guarded):
        wait_gather(j, bi)
        if guarded:
            @pl.when(g > 0)
            def _():
                wait_scatter(j - 2, bi)
        else:
            wait_scatter(j - 2, bi)

        @pl.when(j < CW - 1)
        def _():
            start_gather(j + 1, 1 - bi)
        scale(j, bi)
        start_scatter(j, bi)

    def group(g, carry):
        j0 = 2 * g
        step(j0, 0, g, True)
        step(j0 + 1, 1, g, True)
        return carry

    def chunk_body(chunk, carry):
        pltpu.sync_copy(row_hbm.at[wid, chunk], rowv)
        pltpu.sync_copy(col_hbm.at[wid, chunk], colv)
        pltpu.sync_copy(w_hbm.at[wid, chunk], wv)
        start_gather(0, 0)
        lax.fori_loop(0, (CW - 1) // 2, group, 0)
        wait_gather(CW - 1, 0)
        wait_scatter(CW - 3, 0)
        scale(CW - 1, 0)
        start_scatter(CW - 1, 0)
        wait_scatter(CW - 2, 1)
        wait_scatter(CW - 1, 0)
        return carry

    lax.fori_loop(0, 5, chunk_body, 0)
    plsc.subcore_barrier()
    for i in range(rps // 128):
        pltpu.sync_copy(accsh.at[pl.ds(s * rps + i * 128, 128)],
                        out_hbm.at[c, pl.ds(s * rps + i * 128, 128)])


_B = 1000


def _embed_body(xr, d0r, d1r, gr, br, Wer, ber, W1r, W1pr,
                hws_out, hwsb_out, dis_out):
    deg = d0r[...] + d1r[...] + 1.0
    dis = lax.rsqrt(deg)
    xb = xr[...]
    mu = jnp.mean(xb, axis=1, keepdims=True)
    va = jnp.mean((xb - mu) * (xb - mu), axis=1, keepdims=True)
    ln = (xb - mu) * lax.rsqrt(va + 1e-5) * gr[...] + br[...]
    h0 = jnp.dot(ln, Wer[...], preferred_element_type=jnp.float32) + ber[...]
    hws_out[...] = jnp.dot(h0, W1r[...], preferred_element_type=jnp.float32) * dis
    hwsb_out[...] = (jnp.dot(h0, W1pr[...], preferred_element_type=jnp.float32)
                     * dis).astype(jnp.bfloat16)
    dis_out[...] = dis


def _mid_body(a0r, a1r, hr, disr, br, Wr, Wpr, out, outb):
    dis = disr[...]
    h = jnp.maximum(dis * (a0r[...] + a1r[...] + hr[...]) + br[...], 0.0)
    out[...] = jnp.dot(h, Wr[...], preferred_element_type=jnp.float32) * dis
    outb[...] = (jnp.dot(h, Wpr[...], preferred_element_type=jnp.float32)
                 * dis).astype(jnp.bfloat16)


def _final_body(a0r, a1r, hr, disr, br, xr, batchr, gfr, Wgr, bgr,
                W1av, W1cv, W1gv, bh1r, Wh2r, bh2r, qout,
                sv, sc_, cv, cc_):
    i = pl.program_id(0)

    @pl.when(i == 0)
    def _():
        sv[...] = jnp.zeros_like(sv)
        sc_[...] = jnp.zeros_like(sc_)
        cv[...] = jnp.zeros_like(cv)
        cc_[...] = jnp.zeros_like(cc_)

    dis = disr[...]
    h = jnp.maximum(dis * (a0r[...] + a1r[...] + hr[...]) + br[...], 0.0)
    xb = xr[...]
    mv = (xb[:, H - 2:H - 1] > 0.5).astype(jnp.float32)
    mc = (xb[:, H - 1:H] > 0.5).astype(jnp.float32)
    onehot = (batchr[...] == lax.broadcasted_iota(jnp.int32, (_B, G), 1)
              ).astype(jnp.float32)
    Sv = onehot * mv
    Sc = onehot * mc
    dn = (((0,), (0,)), ((), ()))
    sv[...] += lax.dot_general(Sv, h, dn, preferred_element_type=jnp.float32)
    sc_[...] += lax.dot_general(Sc, h, dn, preferred_element_type=jnp.float32)
    ones = jnp.ones((_B, 1), jnp.float32)
    cv[...] += lax.dot_general(Sv, ones, dn, preferred_element_type=jnp.float32)
    cc_[...] += lax.dot_general(Sc, ones, dn, preferred_element_type=jnp.float32)

    @pl.when(i == pl.num_programs(0) - 1)
    def _():
        ve = sv[...] / jnp.clip(cv[...], 1.0, None)
        ce = sc_[...] / jnp.clip(cc_[...], 1.0, None)
        ge = jnp.maximum(
            jnp.dot(gfr[...], Wgr[...], preferred_element_type=jnp.float32)
            + bgr[...], 0.0)
        comb = jnp.maximum(
            jnp.dot(ve, W1av[...], preferred_element_type=jnp.float32)
            + jnp.dot(ce, W1cv[...], preferred_element_type=jnp.float32)
            + jnp.dot(ge, W1gv[...], preferred_element_type=jnp.float32)
            + bh1r[...], 0.0)
        qout[...] = (jnp.dot(comb, Wh2r[...], preferred_element_type=jnp.float32)
                     + bh2r[...])


def _row_spec(shape):
    return pl.BlockSpec(shape, lambda i: (i, 0))


def _full_spec(shape):
    return pl.BlockSpec(shape, lambda i: (0, 0))


def kernel(x, edge_index, batch, global_features, edge_weight,
           ln_g, ln_b, W_emb, b_emb, W1, b1, W2, b2, W3, b3,
           Wg, bg, Wh1, bh1, Wh2, bh2):
    f32 = jnp.float32
    row2d = edge_index[0].reshape(NW, 5, NWIN // 5, K)
    col2d = edge_index[1].reshape(NW, 5, NWIN // 5, K)
    w2d = edge_weight.reshape(NW, 5, NWIN // 5, K)

    degp = _deg_kernel(edge_index[1].reshape(NW, NWIN, K),
                       edge_weight.reshape(NW, NWIN, K))
    d0 = degp[0, :N].reshape(N, 1)
    d1 = degp[1, :N].reshape(N, 1)

    perm = []
    for u in range(H // 32):
        blk = [0] * 32
        for i in range(16):
            blk[2 * i] = 32 * u + i
            blk[2 * i + 1] = 32 * u + 16 + i
        perm.extend(blk)
    perm = jnp.array(perm, jnp.int32)
    W1p, W2p, W3p = W1[:, perm], W2[:, perm], W3[:, perm]

    grid = N // _B
    hws1, hwsb1, dis = pl.pallas_call(
        _embed_body,
        grid=(grid,),
        in_specs=[_row_spec((_B, H)), _row_spec((_B, 1)), _row_spec((_B, 1)),
                  _full_spec((1, H)), _full_spec((1, H)),
                  _full_spec((H, H)), _full_spec((1, H)), _full_spec((H, H)),
                  _full_spec((H, H))],
        out_specs=[_row_spec((_B, H)), _row_spec((_B, H)), _row_spec((_B, 1))],
        out_shape=[jax.ShapeDtypeStruct((N, H), f32),
                   jax.ShapeDtypeStruct((N, H), jnp.bfloat16),
                   jax.ShapeDtypeStruct((N, 1), f32)],
    )(x, d0, d1, ln_g.reshape(1, H), ln_b.reshape(1, H),
      W_emb, b_emb.reshape(1, H), W1, W1p)

    def pack32(hb):
        return jax.lax.bitcast_convert_type(
            hb.reshape(N, H // 2, 2), jnp.int32)

    def mid(hwsb, hws, b_prev, W_next, Wp_next):
        accp = _scatter_kernel(pack32(hwsb), row2d, col2d, w2d)
        acc = accp[:, :N]
        return pl.pallas_call(
            _mid_body,
            grid=(grid,),
            in_specs=[_row_spec((_B, H)), _row_spec((_B, H)),
                      _row_spec((_B, H)), _row_spec((_B, 1)),
                      _full_spec((1, H)), _full_spec((H, H)),
                      _full_spec((H, H))],
            out_specs=[_row_spec((_B, H)), _row_spec((_B, H))],
            out_shape=[jax.ShapeDtypeStruct((N, H), f32),
                       jax.ShapeDtypeStruct((N, H), jnp.bfloat16)],
        )(acc[0], acc[1], hws, dis, b_prev.reshape(1, H), W_next, Wp_next)

    hws2, hwsb2 = mid(hwsb1, hws1, b1, W2, W2p)
    hws3, hwsb3 = mid(hwsb2, hws2, b2, W3, W3p)
    acc3 = _scatter_kernel(pack32(hwsb3), row2d, col2d, w2d)[:, :N]

    q = pl.pallas_call(
        _final_body,
        grid=(grid,),
        in_specs=[_row_spec((_B, H)), _row_spec((_B, H)), _row_spec((_B, H)),
                  _row_spec((_B, 1)), _full_spec((1, H)), _row_spec((_B, H)),
                  _row_spec((_B, 1)),
                  _full_spec((G, 64)), _full_spec((64, H)), _full_spec((1, H)),
                  _full_spec((H, H)), _full_spec((H, H)), _full_spec((H, H)),
                  _full_spec((1, H)), _full_spec((H, 32)), _full_spec((1, 32))],
        out_specs=_full_spec((G, 32)),
        out_shape=jax.ShapeDtypeStruct((G, 32), f32),
        scratch_shapes=[pltpu.VMEM((G, H), f32), pltpu.VMEM((G, H), f32),
                        pltpu.VMEM((G, 1), f32), pltpu.VMEM((G, 1), f32)],
    )(acc3[0], acc3[1], hws3, dis, b3.reshape(1, H), x,
      batch.reshape(N, 1), global_features, Wg, bg.reshape(1, H),
      Wh1[:H], Wh1[H:2 * H], Wh1[2 * H:], bh1.reshape(1, H), Wh2,
      bh2.reshape(1, 32))
    return q


# speedup vs baseline: 1.1744x; 1.0071x over previous
import functools

import jax
import jax.numpy as jnp
from jax import lax
from jax.experimental import pallas as pl
from jax.experimental.pallas import tpu as pltpu
from jax.experimental.pallas import tpu_sc as plsc

N = 10000
E = 320000
H = 128
G = 16
NC = 2
NS = 16
NW = NC * NS
EPT = E // NW
K = 80
NWIN = EPT // K
RPS = N // NS
NPAD = 10240

_mesh = plsc.VectorSubcoreMesh(core_axis_name="c", subcore_axis_name="s")


@functools.partial(
    pl.kernel,
    mesh=_mesh,
    out_type=jax.ShapeDtypeStruct((NC, NPAD), jnp.float32),
    scratch_types=[
        pltpu.VMEM((NWIN, K), jnp.int32),
        pltpu.VMEM((NWIN, K), jnp.float32),
        pltpu.VMEM((NPAD // NS,), jnp.float32),
        pltpu.VMEM_SHARED((NPAD,), jnp.float32),
    ],
)
def _deg_kernel(col_hbm, w_hbm, out_hbm, colv, wv, zerov, accsh):
    c = lax.axis_index("c")
    s = lax.axis_index("s")
    wid = s * NC + c
    pltpu.sync_copy(col_hbm.at[wid], colv)
    pltpu.sync_copy(w_hbm.at[wid], wv)
    zslice = NPAD // NS
    for u in range(zslice // 16):
        zerov[pl.ds(u * 16, 16)] = jnp.zeros((16,), jnp.float32)
    pltpu.sync_copy(zerov, accsh.at[pl.ds(s * zslice, zslice)])
    plsc.subcore_barrier()

    def body(j, carry):
        pltpu.sync_copy(wv.at[j], accsh.at[colv.at[j]], add=True)
        return carry

    lax.fori_loop(0, NWIN, body, 0)
    plsc.subcore_barrier()
    pltpu.sync_copy(accsh.at[pl.ds(s * zslice, zslice)],
                    out_hbm.at[c, pl.ds(s * zslice, zslice)])


@functools.partial(
    pl.kernel,
    mesh=_mesh,
    out_type=jax.ShapeDtypeStruct((NC, NPAD, H), jnp.float32),
    compiler_params=pltpu.CompilerParams(needs_layout_passes=False,
                                         use_tc_tiling_on_sc=False),
    scratch_types=[
        pltpu.VMEM((NWIN // 5, K), jnp.int32),
        pltpu.VMEM((NWIN // 5, K), jnp.int32),
        pltpu.VMEM((NWIN // 5, K), jnp.float32),
        pltpu.VMEM((2, K, H // 2), jnp.int32),
        pltpu.VMEM((2, K, H), jnp.float32),
        pltpu.VMEM((64, H), jnp.float32),
        pltpu.VMEM_SHARED((NPAD, H), jnp.float32),
        pltpu.SemaphoreType.DMA,
        pltpu.SemaphoreType.DMA,
    ],
)
def _scatter_kernel(hwsb_hbm, row_hbm, col_hbm, w_hbm, out_hbm,
                    rowv, colv, wv, rowsb, rowsf, zeros, accsh, gsem, ssem):
    c = lax.axis_index("c")
    s = lax.axis_index("s")
    wid = s * NC + c
    CW = NWIN // 5
    def zfill(j, carry):
        for u in range(H // 16):
            zeros[j, pl.ds(u * 16, 16)] = jnp.zeros((16,), jnp.float32)
        return carry

    lax.fori_loop(0, 64, zfill, 0)
    rps = NPAD // NS
    for i in range(rps // 64):
        pltpu.sync_copy(zeros, accsh.at[pl.ds(s * rps + i * 64, 64)])
    plsc.subcore_barrier()

    def wait_gather(j, bi):
        pltpu.make_async_copy(hwsb_hbm.at[rowv.at[j]], rowsb.at[bi], gsem).wait()

    def start_gather(j, bi):
        pltpu.async_copy(hwsb_hbm.at[rowv.at[j]], rowsb.at[bi], gsem)

    def wait_scatter(j, bi):
        pltpu.make_async_copy(rowsf.at[bi], accsh.at[colv.at[j]], ssem).wait()

    def start_scatter(j, bi):
        pltpu.async_copy(rowsf.at[bi], accsh.at[colv.at[j]], ssem, add=True)

    def scale(j, bi):
        himask = jnp.full((16,), -65536, jnp.int32)
        for b in range(K // 16):
            wchunk = wv[j, pl.ds(b * 16, 16)]
            for l in range(16):
                i = b * 16 + l
                wj = wchunk[l]
                for u in range(H // 32):
                    v = rowsb[bi, i, pl.ds(u * 16, 16)]
                    lo = plsc.bitcast(v << 16, jnp.float32)
                    hi = plsc.bitcast(v & himask, jnp.float32)
                    rowsf[bi, i, pl.ds(u * 32, 16)] = lo * wj
                    rowsf[bi, i, pl.ds(u * 32 + 16, 16)] = hi * wj

    def step(j, bi, g, guarded):
        wait_gather(j, bi)
        if guarded:
            @pl.when(g > 0)
            def _():
                wait_scatter(j - 2, bi)
        else:
            wait_scatter(j - 2, bi)

        @pl.when(j < CW - 1)
        def _():
            start_gather(j + 1, 1 - bi)
        scale(j, bi)
        start_scatter(j, bi)

    def group(g, carry):
        j0 = 2 * g
        step(j0, 0, g, True)
        step(j0 + 1, 1, g, True)
        return carry

    def chunk_body(chunk, carry):
        pltpu.sync_copy(row_hbm.at[wid, chunk], rowv)
        pltpu.sync_copy(col_hbm.at[wid, chunk], colv)
        pltpu.sync_copy(w_hbm.at[wid, chunk], wv)
        start_gather(0, 0)
        lax.fori_loop(0, (CW - 1) // 2, group, 0)
        wait_gather(CW - 1, 0)
        wait_scatter(CW - 3, 0)
        scale(CW - 1, 0)
        start_scatter(CW - 1, 0)
        wait_scatter(CW - 2, 1)
        wait_scatter(CW - 1, 0)
        return carry

    lax.fori_loop(0, 5, chunk_body, 0)
    plsc.subcore_barrier()
    for i in range(rps // 128):
        pltpu.sync_copy(accsh.at[pl.ds(s * rps + i * 128, 128)],
                        out_hbm.at[c, pl.ds(s * rps + i * 128, 128)])


_B = 1000


def _embed_body(xr, d0r, d1r, gr, br, Wer, ber, W1r, W1pr,
                hws_out, hwsb_out, dis_out):
    deg = d0r[...] + d1r[...] + 1.0
    dis = lax.rsqrt(deg)
    xb = xr[...]
    mu = jnp.mean(xb, axis=1, keepdims=True)
    va = jnp.mean((xb - mu) * (xb - mu), axis=1, keepdims=True)
    ln = (xb - mu) * lax.rsqrt(va + 1e-5) * gr[...] + br[...]
    h0 = jnp.dot(ln, Wer[...], preferred_element_type=jnp.float32) + ber[...]
    hws_out[...] = jnp.dot(h0, W1r[...], preferred_element_type=jnp.float32) * dis
    hwsb_out[...] = (jnp.dot(h0, W1pr[...], preferred_element_type=jnp.float32)
                     * dis).astype(jnp.bfloat16)
    dis_out[...] = dis


def _mid_body(a0r, a1r, hr, disr, br, Wr, Wpr, out, outb):
    dis = disr[...]
    h = jnp.maximum(dis * (a0r[...] + a1r[...] + hr[...]) + br[...], 0.0)
    out[...] = jnp.dot(h, Wr[...], preferred_element_type=jnp.float32) * dis
    outb[...] = (jnp.dot(h, Wpr[...], preferred_element_type=jnp.float32)
                 * dis).astype(jnp.bfloat16)


def _final_body(a0r, a1r, hr, disr, br, xr, batchr, gfr, Wgr, bgr,
                W1av, W1cv, W1gv, bh1r, Wh2r, bh2r, qout,
                sv, sc_, cv, cc_):
    i = pl.program_id(0)

    @pl.when(i == 0)
    def _():
        sv[...] = jnp.zeros_like(sv)
        sc_[...] = jnp.zeros_like(sc_)
        cv[...] = jnp.zeros_like(cv)
        cc_[...] = jnp.zeros_like(cc_)

    dis = disr[...]
    h = jnp.maximum(dis * (a0r[...] + a1r[...] + hr[...]) + br[...], 0.0)
    xb = xr[...]
    mv = (xb[:, H - 2:H - 1] > 0.5).astype(jnp.float32)
    mc = (xb[:, H - 1:H] > 0.5).astype(jnp.float32)
    onehot = (batchr[...] == lax.broadcasted_iota(jnp.int32, (_B, G), 1)
              ).astype(jnp.float32)
    Sv = onehot * mv
    Sc = onehot * mc
    dn = (((0,), (0,)), ((), ()))
    sv[...] += lax.dot_general(Sv, h, dn, preferred_element_type=jnp.float32)
    sc_[...] += lax.dot_general(Sc, h, dn, preferred_element_type=jnp.float32)
    ones = jnp.ones((_B, 1), jnp.float32)
    cv[...] += lax.dot_general(Sv, ones, dn, preferred_element_type=jnp.float32)
    cc_[...] += lax.dot_general(Sc, ones, dn, preferred_element_type=jnp.float32)

    @pl.when(i == pl.num_programs(0) - 1)
    def _():
        ve = sv[...] / jnp.clip(cv[...], 1.0, None)
        ce = sc_[...] / jnp.clip(cc_[...], 1.0, None)
        ge = jnp.maximum(
            jnp.dot(gfr[...], Wgr[...], preferred_element_type=jnp.float32)
            + bgr[...], 0.0)
        comb = jnp.maximum(
            jnp.dot(ve, W1av[...], preferred_element_type=jnp.float32)
            + jnp.dot(ce, W1cv[...], preferred_element_type=jnp.float32)
            + jnp.dot(ge, W1gv[...], preferred_element_type=jnp.float32)
            + bh1r[...], 0.0)
        qout[...] = (jnp.dot(comb, Wh2r[...], preferred_element_type=jnp.float32)
                     + bh2r[...])


def _row_spec(shape):
    return pl.BlockSpec(shape, lambda i: (i, 0))


def _full_spec(shape):
    return pl.BlockSpec(shape, lambda i: (0, 0))


def kernel(x, edge_index, batch, global_features, edge_weight,
           ln_g, ln_b, W_emb, b_emb, W1, b1, W2, b2, W3, b3,
           Wg, bg, Wh1, bh1, Wh2, bh2):
    f32 = jnp.float32
    row2d = edge_index[0].reshape(NW, 5, NWIN // 5, K)
    col2d = edge_index[1].reshape(NW, 5, NWIN // 5, K)
    w2d = edge_weight.reshape(NW, 5, NWIN // 5, K)

    degp = _deg_kernel(edge_index[1].reshape(NW, NWIN, K),
                       edge_weight.reshape(NW, NWIN, K))
    d0 = degp[0, :N].reshape(N, 1)
    d1 = degp[1, :N].reshape(N, 1)

    perm = []
    for u in range(H // 32):
        blk = [0] * 32
        for i in range(16):
            blk[2 * i] = 32 * u + i
            blk[2 * i + 1] = 32 * u + 16 + i
        perm.extend(blk)
    perm = jnp.array(perm, jnp.int32)
    W1p, W2p, W3p = W1[:, perm], W2[:, perm], W3[:, perm]

    grid = N // _B
    hws1, hwsb1, dis = pl.pallas_call(
        _embed_body,
        grid=(grid,),
        in_specs=[_row_spec((_B, H)), _row_spec((_B, 1)), _row_spec((_B, 1)),
                  _full_spec((1, H)), _full_spec((1, H)),
                  _full_spec((H, H)), _full_spec((1, H)), _full_spec((H, H)),
                  _full_spec((H, H))],
        out_specs=[_row_spec((_B, H)), _row_spec((_B, H)), _row_spec((_B, 1))],
        out_shape=[jax.ShapeDtypeStruct((N, H), f32),
                   jax.ShapeDtypeStruct((N, H), jnp.bfloat16),
                   jax.ShapeDtypeStruct((N, 1), f32)],
    )(x, d0, d1, ln_g.reshape(1, H), ln_b.reshape(1, H),
      W_emb, b_emb.reshape(1, H), W1, W1p)

    def pack32(hb):
        return jax.lax.bitcast_convert_type(
            hb.reshape(N, H // 2, 2), jnp.int32)

    def mid(hwsb, hws, b_prev, W_next, Wp_next):
        accp = _scatter_kernel(pack32(hwsb), row2d, col2d, w2d)
        acc = accp[:, :N]
        return pl.pallas_call(
            _mid_body,
            grid=(grid,),
            in_specs=[_row_spec((_B, H)), _row_spec((_B, H)),
                      _row_spec((_B, H)), _row_spec((_B, 1)),
                      _full_spec((1, H)), _full_spec((H, H)),
                      _full_spec((H, H))],
            out_specs=[_row_spec((_B, H)), _row_spec((_B, H))],
            out_shape=[jax.ShapeDtypeStruct((N, H), f32),
                       jax.ShapeDtypeStruct((N, H), jnp.bfloat16)],
        )(acc[0], acc[1], hws, dis, b_prev.reshape(1, H), W_next, Wp_next)

    hws2, hwsb2 = mid(hwsb1, hws1, b1, W2, W2p)
    hws3, hwsb3 = mid(hwsb2, hws2, b2, W3, W3p)
    acc3 = _scatter_kernel(pack32(hwsb3), row2d, col2d, w2d)[:, :N]

    q = pl.pallas_call(
        _final_body,
        grid=(grid,),
        in_specs=[_row_spec((_B, H)), _row_spec((_B, H)), _row_spec((_B, H)),
                  _row_spec((_B, 1)), _full_spec((1, H)), _row_spec((_B, H)),
                  _row_spec((_B, 1)),
                  _full_spec((G, 64)), _full_spec((64, H)), _full_spec((1, H)),
                  _full_spec((H, H)), _full_spec((H, H)), _full_spec((H, H)),
                  _full_spec((1, H)), _full_spec((H, 32)), _full_spec((1, 32))],
        out_specs=_full_spec((G, 32)),
        out_shape=jax.ShapeDtypeStruct((G, 32), f32),
        scratch_shapes=[pltpu.VMEM((G, H), f32), pltpu.VMEM((G, H), f32),
                        pltpu.VMEM((G, 1), f32), pltpu.VMEM((G, 1), f32)],
    )(acc3[0], acc3[1], hws3, dis, b3.reshape(1, H), x,
      batch.reshape(N, 1), global_features, Wg, bg.reshape(1, H),
      Wh1[:H], Wh1[H:2 * H], Wh1[2 * H:], bh1.reshape(1, H), Wh2,
      bh2.reshape(1, 32))
    return q
